# baseline (speedup 1.0000x reference)
import functools

import jax
import jax.numpy as jnp
from jax import lax
from jax.experimental import pallas as pl
from jax.experimental.pallas import tpu as pltpu
from jax.experimental.pallas import tpu_sc as plsc

_CHUNK = 4096


@functools.lru_cache(maxsize=None)
def _build(B, V, D):
    info = plsc.get_sparse_core_info()
    nw = info.num_cores * info.num_subcores
    rows_per_w = D // nw
    n_chunks = B // _CHUNK
    mesh = plsc.VectorSubcoreMesh(core_axis_name="c", subcore_axis_name="s")

    @functools.partial(
        pl.kernel,
        mesh=mesh,
        out_type=jax.ShapeDtypeStruct((D, B), jnp.float32),
        compiler_params=pltpu.CompilerParams(needs_layout_passes=False),
        scratch_types=[
            pltpu.VMEM((B,), jnp.int32),
            pltpu.VMEM((V,), jnp.float32),
            pltpu.VMEM((_CHUNK,), jnp.float32),
            pltpu.SemaphoreType.DMA,
            pltpu.SemaphoreType.DMA,
        ],
    )
    def gather_kernel(idx_hbm, table_hbm, out_hbm, ids_v, row_v, out_v,
                      sem_ids, sem_row):
        wid = lax.axis_index("s") * info.num_cores + lax.axis_index("c")
        ids_cp = pltpu.async_copy(idx_hbm, ids_v, sem_ids)
        pltpu.async_copy(table_hbm.at[wid * rows_per_w], row_v, sem_row)
        ids_cp.wait()

        def rr_body(rr, carry):
            d = wid * rows_per_w + rr
            pltpu.make_async_copy(table_hbm.at[d], row_v, sem_row).wait()

            def c_body(c, carry2):
                @plsc.parallel_loop(0, _CHUNK // 16, unroll=8)
                def body(i):
                    idxv = ids_v[pl.ds(c * _CHUNK + i * 16, 16)]
                    out_v[pl.ds(i * 16, 16)] = plsc.load_gather(row_v, [idxv])

                @pl.when(jnp.logical_and(c == n_chunks - 1,
                                         rr < rows_per_w - 1))
                def _():
                    pltpu.async_copy(table_hbm.at[d + 1], row_v, sem_row)

                pltpu.sync_copy(out_v, out_hbm.at[d, pl.ds(c * _CHUNK, _CHUNK)])
                return carry2

            lax.fori_loop(0, n_chunks, c_body, 0)
            return carry

        lax.fori_loop(0, rows_per_w, rr_body, 0)

    return gather_kernel


def kernel(class_ids, table):
    (B,) = class_ids.shape
    V, D = table.shape
    gather_kernel = _build(B, V, D)
    out_t = gather_kernel(class_ids.astype(jnp.int32), table.T)
    return out_t.T


# speedup vs baseline: 1.0058x; 1.0058x over previous
import functools

import jax
import jax.numpy as jnp
from jax import lax
from jax.experimental import pallas as pl
from jax.experimental.pallas import tpu as pltpu
from jax.experimental.pallas import tpu_sc as plsc

_CHUNK = 8192


@functools.lru_cache(maxsize=None)
def _build(B, V, D):
    info = plsc.get_sparse_core_info()
    nw = info.num_cores * info.num_subcores
    rows_per_w = D // nw
    n_chunks = B // _CHUNK
    mesh = plsc.VectorSubcoreMesh(core_axis_name="c", subcore_axis_name="s")

    @functools.partial(
        pl.kernel,
        mesh=mesh,
        out_type=jax.ShapeDtypeStruct((D, B), jnp.float32),
        compiler_params=pltpu.CompilerParams(needs_layout_passes=False),
        scratch_types=[
            pltpu.VMEM((B,), jnp.int32),
            pltpu.VMEM((V,), jnp.float32),
            pltpu.VMEM((_CHUNK,), jnp.float32),
            pltpu.SemaphoreType.DMA,
            pltpu.SemaphoreType.DMA,
        ],
    )
    def gather_kernel(idx_hbm, table_hbm, out_hbm, ids_v, row_v, out_v,
                      sem_ids, sem_row):
        wid = lax.axis_index("s") * info.num_cores + lax.axis_index("c")
        ids_cp = pltpu.async_copy(idx_hbm, ids_v, sem_ids)
        pltpu.async_copy(table_hbm.at[wid * rows_per_w], row_v, sem_row)
        ids_cp.wait()

        def rr_body(rr, carry):
            d = wid * rows_per_w + rr
            pltpu.make_async_copy(table_hbm.at[d], row_v, sem_row).wait()

            def c_body(c, carry2):
                @plsc.parallel_loop(0, _CHUNK // 16, unroll=8)
                def body(i):
                    idxv = ids_v[pl.ds(c * _CHUNK + i * 16, 16)]
                    out_v[pl.ds(i * 16, 16)] = plsc.load_gather(row_v, [idxv])

                @pl.when(jnp.logical_and(c == n_chunks - 1,
                                         rr < rows_per_w - 1))
                def _():
                    pltpu.async_copy(table_hbm.at[d + 1], row_v, sem_row)

                pltpu.sync_copy(out_v, out_hbm.at[d, pl.ds(c * _CHUNK, _CHUNK)])
                return carry2

            lax.fori_loop(0, n_chunks, c_body, 0)
            return carry

        lax.fori_loop(0, rows_per_w, rr_body, 0)

    return gather_kernel


def kernel(class_ids, table):
    (B,) = class_ids.shape
    V, D = table.shape
    gather_kernel = _build(B, V, D)
    out_t = gather_kernel(class_ids.astype(jnp.int32), table.T)
    return out_t.T
